# Initial kernel scaffold; baseline (speedup 1.0000x reference)
import jax
import jax.numpy as jnp
from jax.experimental import pallas as pl


def kernel(x, edge_index, edge_attr, flat_paths, cu_seqlens, path_edge_feats, path_scalar_feats, src_idx, dst_idx, We1, be1, W11, b11, W12, b12, We2, be2, W21, b21, W22, b22, Wh1, bh1, Wh2, bh2):
    raise NotImplementedError("write your pallas kernel here")


# speedup vs baseline: 2.4807x; 2.4807x over previous
import functools

import jax
import jax.numpy as jnp
from jax import lax
from jax.experimental import pallas as pl
from jax.experimental.pallas import tpu as pltpu
from jax.experimental.pallas import tpu_sc as plsc


def _edge_mm_body(ea_ref, wt_ref, b_ref, out_ref):
    out_ref[...] = (
        jnp.dot(ea_ref[...], wt_ref[...], preferred_element_type=jnp.float32)
        + b_ref[...]
    )


def _edge_mm(edge_attr, wt, b):
    E, ED = edge_attr.shape
    H = wt.shape[1]
    BE = 2000
    grid = E // BE
    return pl.pallas_call(
        _edge_mm_body,
        grid=(grid,),
        in_specs=[
            pl.BlockSpec((BE, ED), lambda i: (i, 0)),
            pl.BlockSpec((ED, H), lambda i: (0, 0)),
            pl.BlockSpec((1, H), lambda i: (0, 0)),
        ],
        out_specs=pl.BlockSpec((BE, H), lambda i: (i, 0)),
        out_shape=jax.ShapeDtypeStruct((E, H), jnp.float32),
    )(edge_attr, wt, b)


def _node_mlp_body(x_ref, a0_ref, a1_ref, w1_ref, b1_ref, w2_ref, b2_ref,
                   out_ref, *, final_relu):
    h = x_ref[...] + a0_ref[...] + a1_ref[...]
    h = jnp.maximum(
        jnp.dot(h, w1_ref[...], preferred_element_type=jnp.float32) + b1_ref[...],
        0.0,
    )
    h = jnp.dot(h, w2_ref[...], preferred_element_type=jnp.float32) + b2_ref[...]
    if final_relu:
        h = jnp.maximum(h, 0.0)
    out_ref[...] = h


def _node_mlp(x, a0, a1, w1t, b1, w2t, b2, final_relu):
    N, D = x.shape
    H = w1t.shape[1]
    BN = 2000
    grid = N // BN
    return pl.pallas_call(
        functools.partial(_node_mlp_body, final_relu=final_relu),
        grid=(grid,),
        in_specs=[
            pl.BlockSpec((BN, D), lambda i: (i, 0)),
            pl.BlockSpec((BN, D), lambda i: (i, 0)),
            pl.BlockSpec((BN, D), lambda i: (i, 0)),
            pl.BlockSpec((D, H), lambda i: (0, 0)),
            pl.BlockSpec((1, H), lambda i: (0, 0)),
            pl.BlockSpec((H, H), lambda i: (0, 0)),
            pl.BlockSpec((1, H), lambda i: (0, 0)),
        ],
        out_specs=pl.BlockSpec((BN, H), lambda i: (i, 0)),
        out_shape=jax.ShapeDtypeStruct((N, H), jnp.float32),
    )(x, a0, a1, w1t, b1, w2t, b2)


@functools.lru_cache(maxsize=None)
def _make_edge_pass(N, E, D):
    info = plsc.get_sparse_core_info()
    NC, NS = info.num_cores, info.num_subcores
    NW = NC * NS
    per_tile = E // NW
    C = 80
    n_chunks = per_tile // C
    assert per_tile % C == 0
    base8 = (N // NS) // 8 * 8
    extra = N - base8 * NS
    mesh = plsc.VectorSubcoreMesh(core_axis_name="c", subcore_axis_name="s")

    def _stripe_chunks(nrows):
        out, off = [], 0
        while off < nrows:
            nr = min(C, nrows - off)
            out.append((off, nr))
            off += nr
        return out

    @functools.partial(
        pl.kernel,
        mesh=mesh,
        out_type=jax.ShapeDtypeStruct((NC, N, D), jnp.float32),
        scratch_types=[
            pltpu.VMEM((C,), jnp.int32),
            pltpu.VMEM((C,), jnp.int32),
            pltpu.VMEM((C, D), jnp.float32),
            pltpu.VMEM((C, D), jnp.float32),
            pltpu.VMEM_SHARED((N, D), jnp.float32),
            pltpu.SemaphoreType.DMA,
        ],
    )
    def edge_pass(x_hbm, e_hbm, src_hbm, dst_hbm, out_hbm,
                  src_v, dst_v, gat_v, e_v, agg_sh, sem):
        c = lax.axis_index("c")
        s = lax.axis_index("s")
        wid = s * NC + c

        def zrow(r, _):
            for j in range(D // 16):
                gat_v[r, pl.ds(j * 16, 16)] = jnp.zeros((16,), jnp.float32)
            return 0
        lax.fori_loop(0, C, zrow, 0)
        row0 = s * base8

        def _emit_init(nrows):
            for off, nr in _stripe_chunks(nrows):
                pltpu.sync_copy(gat_v.at[pl.ds(0, nr), :],
                                agg_sh.at[pl.ds(row0 + off, nr), :])

        @pl.when(s == NS - 1)
        def _():
            _emit_init(base8 + extra)

        @pl.when(s != NS - 1)
        def _():
            _emit_init(base8)

        plsc.subcore_barrier()

        base0 = wid * per_tile

        def chunk(i, _):
            base = base0 + i * C
            pltpu.sync_copy(src_hbm.at[pl.ds(base, C)], src_v)
            pltpu.sync_copy(dst_hbm.at[pl.ds(base, C)], dst_v)
            pltpu.async_copy(x_hbm.at[src_v], gat_v, sem).wait()
            pltpu.sync_copy(e_hbm.at[pl.ds(base, C), :], e_v)

            def mrow(r, _):
                for j in range(D // 16):
                    sl = pl.ds(j * 16, 16)
                    gat_v[r, sl] = jnp.maximum(gat_v[r, sl] + e_v[r, sl], 0.0)
                return 0
            lax.fori_loop(0, C, mrow, 0)
            pltpu.sync_copy(gat_v, agg_sh.at[dst_v], add=True)
            return 0
        lax.fori_loop(0, n_chunks, chunk, 0)

        plsc.subcore_barrier()

        def _emit_read(nrows):
            for off, nr in _stripe_chunks(nrows):
                r0 = row0 + off
                pltpu.sync_copy(agg_sh.at[pl.ds(r0, nr), :],
                                gat_v.at[pl.ds(0, nr), :])
                pltpu.sync_copy(gat_v.at[pl.ds(0, nr), :],
                                out_hbm.at[c, pl.ds(r0, nr), :])

        @pl.when(s == NS - 1)
        def _():
            _emit_read(base8 + extra)

        @pl.when(s != NS - 1)
        def _():
            _emit_read(base8)

    return edge_pass


@functools.lru_cache(maxsize=None)
def _make_path_pool(N, D, P, L):
    info = plsc.get_sparse_core_info()
    NC, NS = info.num_cores, info.num_subcores
    NW = NC * NS
    per_tile = (P * L) // NW
    C = 128
    n_chunks = per_tile // C
    assert per_tile % C == 0
    halves = NW // P
    mesh = plsc.VectorSubcoreMesh(core_axis_name="c", subcore_axis_name="s")

    @functools.partial(
        pl.kernel,
        mesh=mesh,
        out_type=jax.ShapeDtypeStruct((NW * D,), jnp.float32),
        scratch_types=[
            pltpu.VMEM((C,), jnp.int32),
            pltpu.VMEM((C, D), jnp.float32),
            pltpu.VMEM((D,), jnp.float32),
            pltpu.SemaphoreType.DMA,
        ],
    )
    def path_pool(emb_hbm, idx_hbm, out_hbm, idx_v, gat_v, acc_v, sem):
        c = lax.axis_index("c")
        s = lax.axis_index("s")
        w = c * NS + s
        path = w % P
        half = w // P
        base0 = path * L + half * (L // halves)
        for j in range(D // 16):
            acc_v[pl.ds(j * 16, 16)] = jnp.zeros((16,), jnp.float32)

        def chunk(i, _):
            pltpu.sync_copy(idx_hbm.at[pl.ds(base0 + i * C, C)], idx_v)
            pltpu.async_copy(emb_hbm.at[idx_v], gat_v, sem).wait()

            def arow(r, _):
                for j in range(D // 16):
                    sl = pl.ds(j * 16, 16)
                    acc_v[sl] = acc_v[sl] + gat_v[r, sl]
                return 0
            lax.fori_loop(0, C, arow, 0)
            return 0
        lax.fori_loop(0, n_chunks, chunk, 0)
        pltpu.sync_copy(acc_v, out_hbm.at[pl.ds(w * D, D)])

    return path_pool


def _head_body(part_ref, invl_ref, se_ref, de_ref, pe_ref, ps_ref,
               wsrc_ref, wdst_ref, wp_ref, wpe_ref, wps_ref,
               bh1_ref, wh2_ref, bh2_ref, q_ref, *, P):
    path_sum = part_ref[0:P, :] + part_ref[P:2 * P, :]
    path_emb = path_sum * invl_ref[...]
    acc = jnp.dot(se_ref[...], wsrc_ref[...], preferred_element_type=jnp.float32)
    acc = acc + jnp.dot(de_ref[...], wdst_ref[...],
                        preferred_element_type=jnp.float32)
    hq = jnp.dot(path_emb, wp_ref[...], preferred_element_type=jnp.float32)
    hq = hq + jnp.dot(pe_ref[...], wpe_ref[...],
                      preferred_element_type=jnp.float32)
    hq = hq + jnp.dot(ps_ref[...], wps_ref[...],
                      preferred_element_type=jnp.float32)
    hq = jnp.maximum(hq + acc + bh1_ref[...], 0.0)
    q_ref[...] = jnp.sum(hq * wh2_ref[...], axis=1, keepdims=True) + bh2_ref[...]


def _head(part, invl, se, de, pe, ps, wsrc, wdst, wp, wpe, wps, bh1, wh2, bh2):
    P = invl.shape[0]
    return pl.pallas_call(
        functools.partial(_head_body, P=P),
        out_shape=jax.ShapeDtypeStruct((P, 1), jnp.float32),
    )(part, invl, se, de, pe, ps, wsrc, wdst, wp, wpe, wps, bh1, wh2, bh2)


def kernel(x, edge_index, edge_attr, flat_paths, cu_seqlens, path_edge_feats,
           path_scalar_feats, src_idx, dst_idx, We1, be1, W11, b11, W12, b12,
           We2, be2, W21, b21, W22, b22, Wh1, bh1, Wh2, bh2):
    N, D = x.shape
    E = edge_attr.shape[0]
    ED = edge_attr.shape[1]
    P = path_edge_feats.shape[0]
    L = flat_paths.shape[0] // P

    src = edge_index[0]
    dst = edge_index[1]

    e1 = _edge_mm(edge_attr, We1.T, be1[None, :])
    e2 = _edge_mm(edge_attr, We2.T, be2[None, :])

    edge_pass = _make_edge_pass(N, E, D)
    agg1 = edge_pass(x, e1, src, dst)
    h = _node_mlp(x, agg1[0], agg1[1], W11.T, b11[None, :], W12.T, b12[None, :],
                  final_relu=True)
    agg2 = edge_pass(h, e2, src, dst)
    embs = _node_mlp(h, agg2[0], agg2[1], W21.T, b21[None, :], W22.T,
                     b22[None, :], final_relu=False)

    part = _make_path_pool(N, D, P, L)(embs, flat_paths)
    part = part.reshape(2 * P, D)

    lens = (cu_seqlens[1:] - cu_seqlens[:-1]).astype(jnp.float32)
    invl = jnp.broadcast_to((1.0 / jnp.maximum(lens, 1.0))[:, None], (P, D))
    se = lax.dynamic_slice_in_dim(embs, src_idx, 1, 0)
    de = lax.dynamic_slice_in_dim(embs, dst_idx, 1, 0)

    wsrc = Wh1[:, 0:D].T
    wdst = Wh1[:, D:2 * D].T
    wp = Wh1[:, 2 * D:3 * D].T
    wpe = Wh1[:, 3 * D:3 * D + ED].T
    wps = Wh1[:, 3 * D + ED:].T

    q = _head(part, invl, se, de, path_edge_feats, path_scalar_feats,
              wsrc, wdst, wp, wpe, wps, bh1[None, :], Wh2, bh2[None, :])
    return q[:, 0]


# speedup vs baseline: 2.6964x; 1.0870x over previous
import functools

import jax
import jax.numpy as jnp
from jax import lax
from jax.experimental import pallas as pl
from jax.experimental.pallas import tpu as pltpu
from jax.experimental.pallas import tpu_sc as plsc


_DNT = (((1,), (1,)), ((), ()))


def _dott(a, b):
    return jax.lax.dot_general(a, b, _DNT, preferred_element_type=jnp.float32)


def _edge_mm_body(ea_ref, w_ref, b_ref, out_ref):
    out_ref[...] = _dott(ea_ref[...], w_ref[...]) + b_ref[...]


def _edge_mm(edge_attr, w, b):
    E, ED = edge_attr.shape
    H = w.shape[0]
    BE = 2000
    grid = E // BE
    return pl.pallas_call(
        _edge_mm_body,
        grid=(grid,),
        in_specs=[
            pl.BlockSpec((BE, ED), lambda i: (i, 0)),
            pl.BlockSpec((H, ED), lambda i: (0, 0)),
            pl.BlockSpec((1, H), lambda i: (0, 0)),
        ],
        out_specs=pl.BlockSpec((BE, H), lambda i: (i, 0)),
        out_shape=jax.ShapeDtypeStruct((E, H), jnp.float32),
    )(edge_attr, w, b)


def _node_mlp_body(x_ref, a0_ref, a1_ref, w1_ref, b1_ref, w2_ref, b2_ref,
                   out_ref, *, final_relu):
    h = x_ref[...] + a0_ref[...] + a1_ref[...]
    h = jnp.maximum(_dott(h, w1_ref[...]) + b1_ref[...], 0.0)
    h = _dott(h, w2_ref[...]) + b2_ref[...]
    if final_relu:
        h = jnp.maximum(h, 0.0)
    out_ref[...] = h


def _node_mlp(x, a0, a1, w1t, b1, w2t, b2, final_relu):
    N, D = x.shape
    H = w1t.shape[0]
    BN = 2000
    grid = N // BN
    return pl.pallas_call(
        functools.partial(_node_mlp_body, final_relu=final_relu),
        grid=(grid,),
        in_specs=[
            pl.BlockSpec((BN, D), lambda i: (i, 0)),
            pl.BlockSpec((BN, D), lambda i: (i, 0)),
            pl.BlockSpec((BN, D), lambda i: (i, 0)),
            pl.BlockSpec((D, H), lambda i: (0, 0)),
            pl.BlockSpec((1, H), lambda i: (0, 0)),
            pl.BlockSpec((H, H), lambda i: (0, 0)),
            pl.BlockSpec((1, H), lambda i: (0, 0)),
        ],
        out_specs=pl.BlockSpec((BN, H), lambda i: (i, 0)),
        out_shape=jax.ShapeDtypeStruct((N, H), jnp.float32),
    )(x, a0, a1, w1t, b1, w2t, b2)


@functools.lru_cache(maxsize=None)
def _make_edge_pass(N, E, D):
    info = plsc.get_sparse_core_info()
    NC, NS = info.num_cores, info.num_subcores
    NW = NC * NS
    per_tile = E // NW
    C = 80
    n_chunks = per_tile // C
    assert per_tile % C == 0
    base8 = (N // NS) // 8 * 8
    extra = N - base8 * NS
    mesh = plsc.VectorSubcoreMesh(core_axis_name="c", subcore_axis_name="s")

    def _stripe_chunks(nrows):
        out, off = [], 0
        while off < nrows:
            nr = min(C, nrows - off)
            out.append((off, nr))
            off += nr
        return out

    @functools.partial(
        pl.kernel,
        mesh=mesh,
        out_type=jax.ShapeDtypeStruct((NC, N, D), jnp.float32),
        scratch_types=[
            pltpu.VMEM((C,), jnp.int32),
            pltpu.VMEM((C,), jnp.int32),
            pltpu.VMEM((C,), jnp.int32),
            pltpu.VMEM((C,), jnp.int32),
            pltpu.VMEM((2, C, D), jnp.float32),
            pltpu.VMEM((2, C, D), jnp.float32),
            pltpu.VMEM_SHARED((N, D), jnp.float32),
            pltpu.SemaphoreType.DMA,
            pltpu.SemaphoreType.DMA,
            pltpu.SemaphoreType.DMA,
            pltpu.SemaphoreType.DMA,
            pltpu.SemaphoreType.DMA,
            pltpu.SemaphoreType.DMA,
            pltpu.SemaphoreType.DMA,
            pltpu.SemaphoreType.DMA,
            pltpu.SemaphoreType.DMA,
            pltpu.SemaphoreType.DMA,
        ],
    )
    def edge_pass(x_hbm, e_hbm, src_hbm, dst_hbm, out_hbm,
                  src0_v, src1_v, dst0_v, dst1_v, gat_v, e_v, agg_sh,
                  isem0, isem1, dsem0, dsem1, gsem0, gsem1, esem0, esem1,
                  ssem0, ssem1):
        src_v = (src0_v, src1_v)
        dst_v = (dst0_v, dst1_v)
        c = lax.axis_index("c")
        s = lax.axis_index("s")
        wid = s * NC + c
        isem = (isem0, isem1)
        dsem = (dsem0, dsem1)
        gsem = (gsem0, gsem1)
        esem = (esem0, esem1)
        ssem = (ssem0, ssem1)

        @pl.loop(0, C)
        def _(r):
            for j in range(D // 16):
                gat_v[0, r, pl.ds(j * 16, 16)] = jnp.zeros((16,), jnp.float32)
        row0 = s * base8

        def _emit_init(nrows):
            for off, nr in _stripe_chunks(nrows):
                pltpu.sync_copy(gat_v.at[0, pl.ds(0, nr), :],
                                agg_sh.at[pl.ds(row0 + off, nr), :])

        @pl.when(s == NS - 1)
        def _():
            _emit_init(base8 + extra)

        @pl.when(s != NS - 1)
        def _():
            _emit_init(base8)

        plsc.subcore_barrier()

        base0 = wid * per_tile

        def src_load(i, p):
            b = base0 + i * C
            pltpu.async_copy(src_hbm.at[pl.ds(b, C)], src_v[p], isem[p])

        def src_wait(i, p):
            b = base0 + i * C
            pltpu.make_async_copy(src_hbm.at[pl.ds(b, C)], src_v[p],
                                  isem[p]).wait()

        def dst_load(i, p):
            b = base0 + i * C
            pltpu.async_copy(dst_hbm.at[pl.ds(b, C)], dst_v[p], dsem[p])

        def dst_wait(i, p):
            b = base0 + i * C
            pltpu.make_async_copy(dst_hbm.at[pl.ds(b, C)], dst_v[p],
                                  dsem[p]).wait()

        def gather(i, p):
            b = base0 + i * C
            pltpu.async_copy(x_hbm.at[src_v[p]], gat_v.at[p], gsem[p])
            pltpu.async_copy(e_hbm.at[pl.ds(b, C), :], e_v.at[p], esem[p])

        def gather_wait(i, p):
            b = base0 + i * C
            pltpu.make_async_copy(x_hbm.at[src_v[p]], gat_v.at[p],
                                  gsem[p]).wait()
            pltpu.make_async_copy(e_hbm.at[pl.ds(b, C), :], e_v.at[p],
                                  esem[p]).wait()

        def compute(p):
            @pl.loop(0, C, unroll=2)
            def _(r):
                for j in range(D // 16):
                    sl = pl.ds(j * 16, 16)
                    gat_v[p, r, sl] = jnp.maximum(
                        gat_v[p, r, sl] + e_v[p, r, sl], 0.0)

        def scatter(p):
            pltpu.async_copy(gat_v.at[p], agg_sh.at[dst_v[p]], ssem[p],
                             add=True)

        def scatter_wait(p):
            pltpu.make_async_copy(gat_v.at[p], agg_sh.at[dst_v[p]],
                                  ssem[p]).wait()

        def body(i, p):
            gather_wait(i, p)
            scatter_wait(1 - p)
            dst_load(i + 1, 1 - p)
            src_wait(i + 1, 1 - p)
            gather(i + 1, 1 - p)
            src_load(i + 2, p)
            compute(p)
            dst_wait(i, p)
            scatter(p)

        src_load(0, 0)
        dst_load(0, 0)
        src_wait(0, 0)
        gather(0, 0)
        src_load(1, 1)
        gather_wait(0, 0)
        dst_load(1, 1)
        src_wait(1, 1)
        gather(1, 1)
        src_load(2, 0)
        compute(0)
        dst_wait(0, 0)
        scatter(0)

        @pl.loop(0, (n_chunks - 3) // 2)
        def _(k):
            i = 1 + 2 * k
            body(i, 1)
            body(i + 1, 0)

        i = n_chunks - 2
        gather_wait(i, 1)
        scatter_wait(0)
        dst_load(i + 1, 0)
        src_wait(i + 1, 0)
        gather(i + 1, 0)
        compute(1)
        dst_wait(i, 1)
        scatter(1)
        gather_wait(n_chunks - 1, 0)
        scatter_wait(1)
        compute(0)
        dst_wait(n_chunks - 1, 0)
        scatter(0)
        scatter_wait(0)

        plsc.subcore_barrier()

        def _emit_read(nrows):
            for off, nr in _stripe_chunks(nrows):
                r0 = row0 + off
                pltpu.sync_copy(agg_sh.at[pl.ds(r0, nr), :],
                                gat_v.at[0, pl.ds(0, nr), :])
                pltpu.sync_copy(gat_v.at[0, pl.ds(0, nr), :],
                                out_hbm.at[c, pl.ds(r0, nr), :])

        @pl.when(s == NS - 1)
        def _():
            _emit_read(base8 + extra)

        @pl.when(s != NS - 1)
        def _():
            _emit_read(base8)

    return edge_pass


@functools.lru_cache(maxsize=None)
def _make_path_pool(N, D, P, L):
    info = plsc.get_sparse_core_info()
    NC, NS = info.num_cores, info.num_subcores
    NW = NC * NS
    per_tile = (P * L) // NW
    C = 128
    n_chunks = per_tile // C
    assert per_tile % C == 0
    halves = NW // P
    mesh = plsc.VectorSubcoreMesh(core_axis_name="c", subcore_axis_name="s")

    @functools.partial(
        pl.kernel,
        mesh=mesh,
        out_type=jax.ShapeDtypeStruct((NW * D,), jnp.float32),
        scratch_types=[
            pltpu.VMEM((C,), jnp.int32),
            pltpu.VMEM((C, D), jnp.float32),
            pltpu.VMEM((D,), jnp.float32),
            pltpu.SemaphoreType.DMA,
        ],
    )
    def path_pool(emb_hbm, idx_hbm, out_hbm, idx_v, gat_v, acc_v, sem):
        c = lax.axis_index("c")
        s = lax.axis_index("s")
        w = c * NS + s
        path = w % P
        half = w // P
        base0 = path * L + half * (L // halves)
        for j in range(D // 16):
            acc_v[pl.ds(j * 16, 16)] = jnp.zeros((16,), jnp.float32)

        def chunk(i, _):
            pltpu.sync_copy(idx_hbm.at[pl.ds(base0 + i * C, C)], idx_v)
            pltpu.async_copy(emb_hbm.at[idx_v], gat_v, sem).wait()

            def arow(r, _):
                for j in range(D // 16):
                    sl = pl.ds(j * 16, 16)
                    acc_v[sl] = acc_v[sl] + gat_v[r, sl]
                return 0
            lax.fori_loop(0, C, arow, 0)
            return 0
        lax.fori_loop(0, n_chunks, chunk, 0)
        pltpu.sync_copy(acc_v, out_hbm.at[pl.ds(w * D, D)])

    return path_pool


def _head_body(fused_ref, wh1_ref, bh1_ref, wh2_ref, q_ref):
    hq = jnp.maximum(_dott(fused_ref[...], wh1_ref[...]) + bh1_ref[...], 0.0)
    q_ref[...] = _dott(hq, wh2_ref[...])


def _head(fused, wh1, bh1, wh2):
    P = fused.shape[0]
    return pl.pallas_call(
        _head_body,
        out_shape=jax.ShapeDtypeStruct((P, 1), jnp.float32),
    )(fused, wh1, bh1, wh2)


def kernel(x, edge_index, edge_attr, flat_paths, cu_seqlens, path_edge_feats,
           path_scalar_feats, src_idx, dst_idx, We1, be1, W11, b11, W12, b12,
           We2, be2, W21, b21, W22, b22, Wh1, bh1, Wh2, bh2):
    N, D = x.shape
    E = edge_attr.shape[0]
    ED = edge_attr.shape[1]
    P = path_edge_feats.shape[0]
    L = flat_paths.shape[0] // P

    src = edge_index[0]
    dst = edge_index[1]

    e1 = _edge_mm(edge_attr, We1, be1[None, :])
    e2 = _edge_mm(edge_attr, We2, be2[None, :])

    edge_pass = _make_edge_pass(N, E, D)
    agg1 = edge_pass(x, e1, src, dst)
    h = _node_mlp(x, agg1[0], agg1[1], W11, b11[None, :], W12, b12[None, :],
                  final_relu=True)
    agg2 = edge_pass(h, e2, src, dst)
    embs = _node_mlp(h, agg2[0], agg2[1], W21, b21[None, :], W22,
                     b22[None, :], final_relu=False)

    part = _make_path_pool(N, D, P, L)(embs, flat_paths)
    part = part.reshape(2 * P, D)

    lens = (cu_seqlens[1:] - cu_seqlens[:-1]).astype(jnp.float32)
    path_emb = (part[0:P] + part[P:2 * P]) / jnp.maximum(lens, 1.0)[:, None]
    se = jnp.broadcast_to(lax.dynamic_slice_in_dim(embs, src_idx, 1, 0), (P, D))
    de = jnp.broadcast_to(lax.dynamic_slice_in_dim(embs, dst_idx, 1, 0), (P, D))
    fused = jnp.concatenate(
        [se, de, path_emb, path_edge_feats, path_scalar_feats], axis=1)

    q = _head(fused, Wh1, bh1[None, :], Wh2)
    return q[:, 0] + bh2[0]
